# baseline (speedup 1.0000x reference)
import functools

import jax
import jax.numpy as jnp
from jax import lax
from jax.experimental import pallas as pl
from jax.experimental.pallas import tpu as pltpu
from jax.experimental.pallas import tpu_sc as plsc

N_MOLS = 100
MOL_SIZE = 100
N_ATOMS = N_MOLS * MOL_SIZE + 1
MAX_NB = 16
N_BONDS = 160001
ATOM_DIM = 128
BOND_DIM = 16
HID = 256
DEPTH = 3
GRU_H = ATOM_DIM + HID

NW = 32
NA_PAD = 10240
NB_PAD = 163840
BM = 2048


def _tc_matmul_relu(x, w, out_dtype=jnp.float32):
    m, k = x.shape
    _, n = w.shape

    def body(x_ref, w_ref, o_ref):
        r = jax.nn.relu(
            jnp.dot(x_ref[...], w_ref[...], preferred_element_type=jnp.float32))
        o_ref[...] = r.astype(out_dtype)

    return pl.pallas_call(
        body,
        grid=(m // BM,),
        in_specs=[
            pl.BlockSpec((BM, k), lambda i: (i, 0)),
            pl.BlockSpec((k, n), lambda i: (0, 0)),
        ],
        out_specs=pl.BlockSpec((BM, n), lambda i: (i, 0)),
        out_shape=jax.ShapeDtypeStruct((m, n), out_dtype),
    )(x, w)


def _tc_proj_bond(x, w):
    m, k = x.shape
    _, n = w.shape

    def body(x_ref, w_ref, o32_ref, o16_ref):
        r = jax.nn.relu(
            jnp.dot(x_ref[...], w_ref[...], preferred_element_type=jnp.float32))
        o32_ref[...] = r
        o16_ref[...] = r.astype(jnp.bfloat16)

    return pl.pallas_call(
        body,
        grid=(m // BM,),
        in_specs=[
            pl.BlockSpec((BM, k), lambda i: (i, 0)),
            pl.BlockSpec((k, n), lambda i: (0, 0)),
        ],
        out_specs=[
            pl.BlockSpec((BM, n), lambda i: (i, 0)),
            pl.BlockSpec((BM, n), lambda i: (i, 0)),
        ],
        out_shape=[
            jax.ShapeDtypeStruct((m, n), jnp.float32),
            jax.ShapeDtypeStruct((m, n), jnp.bfloat16),
        ],
    )(x, w)


def _tc_update(delta, ib, w):
    m, k = delta.shape
    _, n = w.shape

    def body(d_ref, ib_ref, w_ref, o_ref):
        o_ref[...] = jax.nn.relu(
            ib_ref[...]
            + jnp.dot(d_ref[...], w_ref[...], preferred_element_type=jnp.float32))

    return pl.pallas_call(
        body,
        grid=(m // BM,),
        in_specs=[
            pl.BlockSpec((BM, k), lambda i: (i, 0)),
            pl.BlockSpec((BM, n), lambda i: (i, 0)),
            pl.BlockSpec((k, n), lambda i: (0, 0)),
        ],
        out_specs=pl.BlockSpec((BM, n), lambda i: (i, 0)),
        out_shape=jax.ShapeDtypeStruct((m, n), jnp.float32),
    )(delta, ib, w)


_CA = 8
_CB = 64


_FAST_CORE = 0
_AGG_SPLIT = (432, 208)
_CMB_SPLIT = (6912, 3328)


def _sc_aggregate(mb, a2b_flat):
    per_s = NA_PAD // 16
    g_rows = _CA * MAX_NB
    len_f, len_s = _AGG_SPLIT
    mesh = plsc.VectorSubcoreMesh(core_axis_name="c", subcore_axis_name="s")

    @functools.partial(
        pl.kernel, mesh=mesh,
        out_type=jax.ShapeDtypeStruct((NA_PAD, HID), jnp.float32),
        scratch_types=[
            pltpu.VMEM((per_s * MAX_NB,), jnp.int32),
            pltpu.VMEM((2, g_rows, HID), jnp.float32),
            pltpu.VMEM((_CA, HID), jnp.float32),
            pltpu.SemaphoreType.DMA,
            pltpu.SemaphoreType.DMA,
        ],
    )
    def k(mb_hbm, a2b_hbm, agg_hbm, a2b_l, rows_v, outb_v, sem0, sem1):
        s_idx = lax.axis_index("s")
        c_idx = lax.axis_index("c")
        base_s = s_idx * per_s
        sems = (sem0, sem1)
        pltpu.sync_copy(a2b_hbm.at[pl.ds(base_s * MAX_NB, per_s * MAX_NB)],
                        a2b_l)

        is_fast = c_idx == _FAST_CORE
        off = jnp.where(is_fast, 0, len_f)
        n_chunks = jnp.where(is_fast, len_f // _CA, len_s // _CA)
        n_pairs = jnp.where(is_fast, len_f // (2 * _CA), len_s // (2 * _CA))

        def gstart(c, b):
            lo = (off + c * _CA) * MAX_NB
            pltpu.async_copy(mb_hbm.at[a2b_l.at[pl.ds(lo, g_rows)]],
                             rows_v.at[b], sems[b])

        def gwait(b):
            pltpu.make_async_copy(mb_hbm.at[a2b_l.at[pl.ds(0, g_rows)]],
                                  rows_v.at[b], sems[b]).wait()

        def compute_wb(c, b):
            rb = rows_v.at[b]
            for a in range(_CA):
                r0 = a * MAX_NB
                for col in range(HID // 16):
                    cs = pl.ds(col * 16, 16)
                    v = rb[r0, cs]
                    s = v
                    mx = v
                    for r in range(1, MAX_NB):
                        v = rb[r0 + r, cs]
                        s = s + v
                        mx = jnp.maximum(mx, v)
                    outb_v[a, cs] = s * mx
            pltpu.sync_copy(
                outb_v, agg_hbm.at[pl.ds(base_s + off + c * _CA, _CA)])

        gstart(0, 0)

        def pair_body(p, carry):
            c0 = p * 2
            gstart(c0 + 1, 1)
            gwait(0)
            compute_wb(c0, 0)

            @pl.when(c0 + 2 < n_chunks)
            def _():
                gstart(c0 + 2, 0)

            gwait(1)
            compute_wb(c0 + 1, 1)
            return carry

        lax.fori_loop(0, n_pairs, pair_body, 0)

    return k(mb, a2b_flat)


def _sc_combine(agg, mb, b2a, b2revb):
    per_s = NB_PAD // 16
    len_f, len_s = _CMB_SPLIT
    mesh = plsc.VectorSubcoreMesh(core_axis_name="c", subcore_axis_name="s")

    @functools.partial(
        pl.kernel, mesh=mesh,
        out_type=jax.ShapeDtypeStruct((NB_PAD, HID), jnp.float32),
        scratch_types=[
            pltpu.VMEM((per_s,), jnp.int32),
            pltpu.VMEM((per_s,), jnp.int32),
            pltpu.VMEM((2, _CB, HID), jnp.float32),
            pltpu.VMEM((2, _CB, HID), jnp.float32),
            pltpu.SemaphoreType.DMA,
            pltpu.SemaphoreType.DMA,
            pltpu.SemaphoreType.DMA,
            pltpu.SemaphoreType.DMA,
        ],
    )
    def k(agg_hbm, mb_hbm, b2a_hbm, b2revb_hbm, out_hbm,
          b2a_l, b2revb_l, g1_v, g2_v, s1a, s1b, s2a, s2b):
        s_idx = lax.axis_index("s")
        c_idx = lax.axis_index("c")
        base_s = s_idx * per_s
        s1 = (s1a, s1b)
        s2 = (s2a, s2b)
        pltpu.sync_copy(b2a_hbm.at[pl.ds(base_s, per_s)], b2a_l)
        pltpu.sync_copy(b2revb_hbm.at[pl.ds(base_s, per_s)], b2revb_l)

        is_fast = c_idx == _FAST_CORE
        off = jnp.where(is_fast, 0, len_f)
        n_chunks = jnp.where(is_fast, len_f // _CB, len_s // _CB)
        n_pairs = jnp.where(is_fast, len_f // (2 * _CB), len_s // (2 * _CB))

        def gstart(c, b):
            sl = pl.ds(off + c * _CB, _CB)
            pltpu.async_copy(agg_hbm.at[b2a_l.at[sl]], g1_v.at[b], s1[b])
            pltpu.async_copy(mb_hbm.at[b2revb_l.at[sl]], g2_v.at[b], s2[b])

        def gwait(b):
            sl = pl.ds(0, _CB)
            pltpu.make_async_copy(agg_hbm.at[b2a_l.at[sl]], g1_v.at[b],
                                  s1[b]).wait()
            pltpu.make_async_copy(mb_hbm.at[b2revb_l.at[sl]], g2_v.at[b],
                                  s2[b]).wait()

        def compute_wb(c, b):
            gb1 = g1_v.at[b]
            gb2 = g2_v.at[b]
            for r in range(_CB):
                for col in range(HID // 16):
                    cs = pl.ds(col * 16, 16)
                    gb1[r, cs] = gb1[r, cs] - gb2[r, cs]
            pltpu.sync_copy(gb1,
                            out_hbm.at[pl.ds(base_s + off + c * _CB, _CB)])

        gstart(0, 0)

        def pair_body(p, carry):
            c0 = p * 2
            gstart(c0 + 1, 1)
            gwait(0)
            compute_wb(c0, 0)

            @pl.when(c0 + 2 < n_chunks)
            def _():
                gstart(c0 + 2, 0)

            gwait(1)
            compute_wb(c0 + 1, 1)
            return carry

        lax.fori_loop(0, n_pairs, pair_body, 0)

    return k(agg, mb, b2a, b2revb)


def _tc_node(agg1, ia1, fa1, gb):
    def body(a_ref, i_ref, f_ref, gb_ref, msg_ref, h0_ref):
        a = a_ref[0].astype(jnp.float32)
        node = jnp.concatenate([a * i_ref[0], f_ref[0]], axis=1)
        msg_ref[...] = jax.nn.relu(node + gb_ref[...]).reshape(1, MOL_SIZE, GRU_H)
        h0_ref[...] = jnp.max(node, axis=0).reshape(1, 1, GRU_H)

    return pl.pallas_call(
        body,
        grid=(N_MOLS,),
        in_specs=[
            pl.BlockSpec((1, MOL_SIZE, HID), lambda m: (m, 0, 0)),
            pl.BlockSpec((1, MOL_SIZE, HID), lambda m: (m, 0, 0)),
            pl.BlockSpec((1, MOL_SIZE, ATOM_DIM), lambda m: (m, 0, 0)),
            pl.BlockSpec((1, GRU_H), lambda m: (0, 0)),
        ],
        out_specs=[
            pl.BlockSpec((1, MOL_SIZE, GRU_H), lambda m: (m, 0, 0)),
            pl.BlockSpec((1, 1, GRU_H), lambda m: (m, 0, 0)),
        ],
        out_shape=[
            jax.ShapeDtypeStruct((N_MOLS, MOL_SIZE, GRU_H), jnp.float32),
            jax.ShapeDtypeStruct((N_MOLS, 1, GRU_H), jnp.float32),
        ],
    )(agg1, ia1, fa1, gb)


def _tc_gru(msgT, h0, wihT, whhT, bih, bhh, wo, bo, w1, b1, w2, b2):
    def body(x_ref, h0_ref, wih_ref, whh_ref, bih_ref, bhh_ref, wo_ref, bo_ref,
             w1_ref, b1_ref, w2_ref, b2_ref, out_ref, h_ref, mv_ref):
        t = pl.program_id(0)

        @pl.when(t == 0)
        def _():
            h_ref[...] = h0_ref[...]

        x = x_ref[0]
        h = h_ref[...]
        gx = jnp.dot(x, wih_ref[...], preferred_element_type=jnp.float32) + bih_ref[...]
        gh = jnp.dot(h, whh_ref[...], preferred_element_type=jnp.float32) + bhh_ref[...]
        r = jax.nn.sigmoid(gx[:, :GRU_H] + gh[:, :GRU_H])
        z = jax.nn.sigmoid(gx[:, GRU_H:2 * GRU_H] + gh[:, GRU_H:2 * GRU_H])
        n = jnp.tanh(gx[:, 2 * GRU_H:] + r * gh[:, 2 * GRU_H:])
        h_new = (1.0 - z) * n + z * h
        h_ref[...] = h_new
        a = jax.nn.relu(
            jnp.dot(h_new, wo_ref[...], preferred_element_type=jnp.float32)
            + bo_ref[...])

        @pl.when(t == 0)
        def _():
            mv_ref[...] = a

        @pl.when(t > 0)
        def _():
            mv_ref[...] = jnp.maximum(mv_ref[...], a)

        @pl.when(t == N_MOLS - 1)
        def _():
            hf = jax.nn.relu(
                jnp.dot(mv_ref[...], w1_ref[...], preferred_element_type=jnp.float32)
                + b1_ref[...])
            out_ref[...] = (
                jnp.dot(hf, w2_ref[...], preferred_element_type=jnp.float32)
                + b2_ref[...])

    return pl.pallas_call(
        body,
        grid=(MOL_SIZE,),
        in_specs=[
            pl.BlockSpec((1, N_MOLS, GRU_H), lambda t: (t, 0, 0)),
            pl.BlockSpec((N_MOLS, GRU_H), lambda t: (0, 0)),
            pl.BlockSpec((GRU_H, 3 * GRU_H), lambda t: (0, 0)),
            pl.BlockSpec((GRU_H, 3 * GRU_H), lambda t: (0, 0)),
            pl.BlockSpec((1, 3 * GRU_H), lambda t: (0, 0)),
            pl.BlockSpec((1, 3 * GRU_H), lambda t: (0, 0)),
            pl.BlockSpec((GRU_H, HID), lambda t: (0, 0)),
            pl.BlockSpec((1, HID), lambda t: (0, 0)),
            pl.BlockSpec((HID, HID), lambda t: (0, 0)),
            pl.BlockSpec((1, HID), lambda t: (0, 0)),
            pl.BlockSpec((HID, 1), lambda t: (0, 0)),
            pl.BlockSpec((1, 1), lambda t: (0, 0)),
        ],
        out_specs=pl.BlockSpec((N_MOLS, 1), lambda t: (0, 0)),
        out_shape=jax.ShapeDtypeStruct((N_MOLS, 1), jnp.float32),
        scratch_shapes=[
            pltpu.VMEM((N_MOLS, GRU_H), jnp.float32),
            pltpu.VMEM((N_MOLS, HID), jnp.float32),
        ],
    )(msgT, h0, wihT, whhT, bih, bhh, wo, bo, w1, b1, w2, b2)


def kernel(f_atoms, f_bonds, a2b, b2a, b2revb, W_i_atom, W_i_bond, W_h_bond,
           W_o, b_o, gru_bias, W_ih, W_hh, b_ih, b_hh, W1, b1, W2, b2):
    f32 = jnp.float32
    fa_p = jnp.pad(f_atoms, ((0, NA_PAD - N_ATOMS), (0, 0)))
    fb_p = jnp.pad(f_bonds, ((0, NB_PAD - N_BONDS), (0, 0)))
    a2b_flat = jnp.pad(a2b.astype(jnp.int32),
                       ((0, NA_PAD - N_ATOMS), (0, 0))).reshape(-1)
    b2a_p = jnp.pad(b2a.astype(jnp.int32), (0, NB_PAD - N_BONDS))
    b2revb_p = jnp.pad(b2revb.astype(jnp.int32), (0, NB_PAD - N_BONDS))

    input_atom = _tc_matmul_relu(fa_p, W_i_atom.astype(f32))
    input_bond = _tc_matmul_relu(fb_p, W_i_bond.astype(f32))

    mb = input_bond
    for _ in range(DEPTH - 1):
        aggp = _sc_aggregate(mb, a2b_flat)
        delta = _sc_combine(aggp, mb, b2a_p, b2revb_p)
        mb = _tc_update(delta, input_bond, W_h_bond.astype(f32))
    aggp = _sc_aggregate(mb, a2b_flat)

    agg1 = aggp[1:N_ATOMS].reshape(N_MOLS, MOL_SIZE, HID)
    ia1 = input_atom[1:N_ATOMS].reshape(N_MOLS, MOL_SIZE, HID)
    fa1 = f_atoms[1:N_ATOMS].reshape(N_MOLS, MOL_SIZE, ATOM_DIM)
    gb = gru_bias.astype(f32).reshape(1, GRU_H)

    msg_seq, h0 = _tc_node(agg1, ia1, fa1, gb)
    msgT = jnp.swapaxes(msg_seq, 0, 1)
    h0 = h0.reshape(N_MOLS, GRU_H)

    out = _tc_gru(
        msgT, h0,
        W_ih.astype(f32).T, W_hh.astype(f32).T,
        b_ih.astype(f32).reshape(1, -1), b_hh.astype(f32).reshape(1, -1),
        W_o.astype(f32), b_o.astype(f32).reshape(1, -1),
        W1.astype(f32), b1.astype(f32).reshape(1, -1),
        W2.astype(f32), b2.astype(f32).reshape(1, -1))
    return out


# speedup vs baseline: 1.3038x; 1.3038x over previous
import functools

import jax
import jax.numpy as jnp
from jax import lax
from jax.experimental import pallas as pl
from jax.experimental.pallas import tpu as pltpu
from jax.experimental.pallas import tpu_sc as plsc

N_MOLS = 100
MOL_SIZE = 100
N_ATOMS = N_MOLS * MOL_SIZE + 1
MAX_NB = 16
N_BONDS = 160001
ATOM_DIM = 128
BOND_DIM = 16
HID = 256
DEPTH = 3
GRU_H = ATOM_DIM + HID

NW = 32
NA_PAD = 10240
NB_PAD = 163840
BM = 2048


def _tc_matmul_relu(x, w, out_dtype=jnp.float32):
    m, k = x.shape
    _, n = w.shape

    def body(x_ref, w_ref, o_ref):
        r = jax.nn.relu(
            jnp.dot(x_ref[...], w_ref[...], preferred_element_type=jnp.float32))
        o_ref[...] = r.astype(out_dtype)

    return pl.pallas_call(
        body,
        grid=(m // BM,),
        in_specs=[
            pl.BlockSpec((BM, k), lambda i: (i, 0)),
            pl.BlockSpec((k, n), lambda i: (0, 0)),
        ],
        out_specs=pl.BlockSpec((BM, n), lambda i: (i, 0)),
        out_shape=jax.ShapeDtypeStruct((m, n), out_dtype),
    )(x, w)


def _tc_proj_bond(x, w):
    m, k = x.shape
    _, n = w.shape

    def body(x_ref, w_ref, o32_ref, o16_ref):
        r = jax.nn.relu(
            jnp.dot(x_ref[...], w_ref[...], preferred_element_type=jnp.float32))
        o32_ref[...] = r
        o16_ref[...] = r.astype(jnp.bfloat16)

    return pl.pallas_call(
        body,
        grid=(m // BM,),
        in_specs=[
            pl.BlockSpec((BM, k), lambda i: (i, 0)),
            pl.BlockSpec((k, n), lambda i: (0, 0)),
        ],
        out_specs=[
            pl.BlockSpec((BM, n), lambda i: (i, 0)),
            pl.BlockSpec((BM, n), lambda i: (i, 0)),
        ],
        out_shape=[
            jax.ShapeDtypeStruct((m, n), jnp.float32),
            jax.ShapeDtypeStruct((m, n), jnp.bfloat16),
        ],
    )(x, w)


def _tc_update(delta, ib, w):
    m, k = delta.shape
    _, n = w.shape

    def body(d_ref, ib_ref, w_ref, o_ref):
        o_ref[...] = jax.nn.relu(
            ib_ref[...]
            + jnp.dot(d_ref[...], w_ref[...], preferred_element_type=jnp.float32))

    return pl.pallas_call(
        body,
        grid=(m // BM,),
        in_specs=[
            pl.BlockSpec((BM, k), lambda i: (i, 0)),
            pl.BlockSpec((BM, n), lambda i: (i, 0)),
            pl.BlockSpec((k, n), lambda i: (0, 0)),
        ],
        out_specs=pl.BlockSpec((BM, n), lambda i: (i, 0)),
        out_shape=jax.ShapeDtypeStruct((m, n), jnp.float32),
    )(delta, ib, w)


_CA = 8
_CB = 64


_FAST_CORE = 0
_AGG_SPLIT = (432, 208)
_CMB_SPLIT = (6912, 3328)


def _sc_aggregate(mb, a2b_flat):
    per_s = NA_PAD // 16
    g_rows = _CA * MAX_NB
    len_f, len_s = _AGG_SPLIT
    mesh = plsc.VectorSubcoreMesh(core_axis_name="c", subcore_axis_name="s")

    @functools.partial(
        pl.kernel, mesh=mesh,
        out_type=jax.ShapeDtypeStruct((NA_PAD, HID), jnp.float32),
        scratch_types=[
            pltpu.VMEM((per_s * MAX_NB,), jnp.int32),
            pltpu.VMEM((2, g_rows, HID), jnp.float32),
            pltpu.VMEM((_CA, HID), jnp.float32),
            pltpu.SemaphoreType.DMA,
            pltpu.SemaphoreType.DMA,
        ],
    )
    def k(mb_hbm, a2b_hbm, agg_hbm, a2b_l, rows_v, outb_v, sem0, sem1):
        s_idx = lax.axis_index("s")
        c_idx = lax.axis_index("c")
        base_s = s_idx * per_s
        sems = (sem0, sem1)
        pltpu.sync_copy(a2b_hbm.at[pl.ds(base_s * MAX_NB, per_s * MAX_NB)],
                        a2b_l)

        def run(off, n_atoms):
            n_chunks = n_atoms // _CA

            def gstart(c, b):
                lo = (off + c * _CA) * MAX_NB
                pltpu.async_copy(mb_hbm.at[a2b_l.at[pl.ds(lo, g_rows)]],
                                 rows_v.at[b], sems[b])

            def gwait(b):
                pltpu.make_async_copy(mb_hbm.at[a2b_l.at[pl.ds(0, g_rows)]],
                                      rows_v.at[b], sems[b]).wait()

            def compute_wb(c, b):
                rb = rows_v.at[b]

                def atom_body(a, acc):
                    r0 = a * MAX_NB
                    for col in range(HID // 16):
                        cs = pl.ds(col * 16, 16)
                        v = rb[r0, cs]
                        s = v
                        mx = v
                        for r in range(1, MAX_NB):
                            v = rb[r0 + r, cs]
                            s = s + v
                            mx = jnp.maximum(mx, v)
                        outb_v[a, cs] = s * mx
                    return acc

                lax.fori_loop(0, _CA, atom_body, 0)
                pltpu.sync_copy(
                    outb_v, agg_hbm.at[pl.ds(base_s + off + c * _CA, _CA)])

            gstart(0, 0)

            def pair_body(p, carry):
                c0 = p * 2
                gstart(c0 + 1, 1)
                gwait(0)
                compute_wb(c0, 0)

                @pl.when(c0 + 2 < n_chunks)
                def _():
                    gstart(c0 + 2, 0)

                gwait(1)
                compute_wb(c0 + 1, 1)
                return carry

            lax.fori_loop(0, n_chunks // 2, pair_body, 0)

        @pl.when(c_idx == _FAST_CORE)
        def _():
            run(0, len_f)

        @pl.when(c_idx != _FAST_CORE)
        def _():
            run(len_f, len_s)

    return k(mb, a2b_flat)


def _sc_combine(agg, mb, b2a, b2revb):
    per_s = NB_PAD // 16
    len_f, len_s = _CMB_SPLIT
    mesh = plsc.VectorSubcoreMesh(core_axis_name="c", subcore_axis_name="s")

    @functools.partial(
        pl.kernel, mesh=mesh,
        out_type=jax.ShapeDtypeStruct((NB_PAD, HID), jnp.float32),
        scratch_types=[
            pltpu.VMEM((per_s,), jnp.int32),
            pltpu.VMEM((per_s,), jnp.int32),
            pltpu.VMEM((2, _CB, HID), jnp.float32),
            pltpu.VMEM((2, _CB, HID), jnp.float32),
            pltpu.SemaphoreType.DMA,
            pltpu.SemaphoreType.DMA,
            pltpu.SemaphoreType.DMA,
            pltpu.SemaphoreType.DMA,
        ],
    )
    def k(agg_hbm, mb_hbm, b2a_hbm, b2revb_hbm, out_hbm,
          b2a_l, b2revb_l, g1_v, g2_v, s1a, s1b, s2a, s2b):
        s_idx = lax.axis_index("s")
        c_idx = lax.axis_index("c")
        base_s = s_idx * per_s
        s1 = (s1a, s1b)
        s2 = (s2a, s2b)
        pltpu.sync_copy(b2a_hbm.at[pl.ds(base_s, per_s)], b2a_l)
        pltpu.sync_copy(b2revb_hbm.at[pl.ds(base_s, per_s)], b2revb_l)

        def run(off, n_bonds):
            n_chunks = n_bonds // _CB

            def gstart(c, b):
                sl = pl.ds(off + c * _CB, _CB)
                pltpu.async_copy(agg_hbm.at[b2a_l.at[sl]], g1_v.at[b], s1[b])
                pltpu.async_copy(mb_hbm.at[b2revb_l.at[sl]], g2_v.at[b], s2[b])

            def gwait(b):
                sl = pl.ds(0, _CB)
                pltpu.make_async_copy(agg_hbm.at[b2a_l.at[sl]], g1_v.at[b],
                                      s1[b]).wait()
                pltpu.make_async_copy(mb_hbm.at[b2revb_l.at[sl]], g2_v.at[b],
                                      s2[b]).wait()

            def compute_wb(c, b):
                gb1 = g1_v.at[b]
                gb2 = g2_v.at[b]

                def row_body(r, acc):
                    for col in range(HID // 16):
                        cs = pl.ds(col * 16, 16)
                        gb1[r, cs] = gb1[r, cs] - gb2[r, cs]
                    return acc

                lax.fori_loop(0, _CB, row_body, 0)
                pltpu.sync_copy(gb1,
                                out_hbm.at[pl.ds(base_s + off + c * _CB, _CB)])

            gstart(0, 0)

            def pair_body(p, carry):
                c0 = p * 2
                gstart(c0 + 1, 1)
                gwait(0)
                compute_wb(c0, 0)

                @pl.when(c0 + 2 < n_chunks)
                def _():
                    gstart(c0 + 2, 0)

                gwait(1)
                compute_wb(c0 + 1, 1)
                return carry

            lax.fori_loop(0, n_chunks // 2, pair_body, 0)

        @pl.when(c_idx == _FAST_CORE)
        def _():
            run(0, len_f)

        @pl.when(c_idx != _FAST_CORE)
        def _():
            run(len_f, len_s)

    return k(agg, mb, b2a, b2revb)


def _tc_node(agg1, ia1, fa1, gb):
    def body(a_ref, i_ref, f_ref, gb_ref, msg_ref, h0_ref):
        a = a_ref[0].astype(jnp.float32)
        node = jnp.concatenate([a * i_ref[0], f_ref[0]], axis=1)
        msg_ref[...] = jax.nn.relu(node + gb_ref[...]).reshape(1, MOL_SIZE, GRU_H)
        h0_ref[...] = jnp.max(node, axis=0).reshape(1, 1, GRU_H)

    return pl.pallas_call(
        body,
        grid=(N_MOLS,),
        in_specs=[
            pl.BlockSpec((1, MOL_SIZE, HID), lambda m: (m, 0, 0)),
            pl.BlockSpec((1, MOL_SIZE, HID), lambda m: (m, 0, 0)),
            pl.BlockSpec((1, MOL_SIZE, ATOM_DIM), lambda m: (m, 0, 0)),
            pl.BlockSpec((1, GRU_H), lambda m: (0, 0)),
        ],
        out_specs=[
            pl.BlockSpec((1, MOL_SIZE, GRU_H), lambda m: (m, 0, 0)),
            pl.BlockSpec((1, 1, GRU_H), lambda m: (m, 0, 0)),
        ],
        out_shape=[
            jax.ShapeDtypeStruct((N_MOLS, MOL_SIZE, GRU_H), jnp.float32),
            jax.ShapeDtypeStruct((N_MOLS, 1, GRU_H), jnp.float32),
        ],
    )(agg1, ia1, fa1, gb)


def _tc_gru(msgT, h0, wihT, whhT, bih, bhh, wo, bo, w1, b1, w2, b2):
    def body(x_ref, h0_ref, wih_ref, whh_ref, bih_ref, bhh_ref, wo_ref, bo_ref,
             w1_ref, b1_ref, w2_ref, b2_ref, out_ref, h_ref, mv_ref):
        t = pl.program_id(0)

        @pl.when(t == 0)
        def _():
            h_ref[...] = h0_ref[...]

        x = x_ref[0]
        h = h_ref[...]
        gx = jnp.dot(x, wih_ref[...], preferred_element_type=jnp.float32) + bih_ref[...]
        gh = jnp.dot(h, whh_ref[...], preferred_element_type=jnp.float32) + bhh_ref[...]
        r = jax.nn.sigmoid(gx[:, :GRU_H] + gh[:, :GRU_H])
        z = jax.nn.sigmoid(gx[:, GRU_H:2 * GRU_H] + gh[:, GRU_H:2 * GRU_H])
        n = jnp.tanh(gx[:, 2 * GRU_H:] + r * gh[:, 2 * GRU_H:])
        h_new = (1.0 - z) * n + z * h
        h_ref[...] = h_new
        a = jax.nn.relu(
            jnp.dot(h_new, wo_ref[...], preferred_element_type=jnp.float32)
            + bo_ref[...])

        @pl.when(t == 0)
        def _():
            mv_ref[...] = a

        @pl.when(t > 0)
        def _():
            mv_ref[...] = jnp.maximum(mv_ref[...], a)

        @pl.when(t == N_MOLS - 1)
        def _():
            hf = jax.nn.relu(
                jnp.dot(mv_ref[...], w1_ref[...], preferred_element_type=jnp.float32)
                + b1_ref[...])
            out_ref[...] = (
                jnp.dot(hf, w2_ref[...], preferred_element_type=jnp.float32)
                + b2_ref[...])

    return pl.pallas_call(
        body,
        grid=(MOL_SIZE,),
        in_specs=[
            pl.BlockSpec((1, N_MOLS, GRU_H), lambda t: (t, 0, 0)),
            pl.BlockSpec((N_MOLS, GRU_H), lambda t: (0, 0)),
            pl.BlockSpec((GRU_H, 3 * GRU_H), lambda t: (0, 0)),
            pl.BlockSpec((GRU_H, 3 * GRU_H), lambda t: (0, 0)),
            pl.BlockSpec((1, 3 * GRU_H), lambda t: (0, 0)),
            pl.BlockSpec((1, 3 * GRU_H), lambda t: (0, 0)),
            pl.BlockSpec((GRU_H, HID), lambda t: (0, 0)),
            pl.BlockSpec((1, HID), lambda t: (0, 0)),
            pl.BlockSpec((HID, HID), lambda t: (0, 0)),
            pl.BlockSpec((1, HID), lambda t: (0, 0)),
            pl.BlockSpec((HID, 1), lambda t: (0, 0)),
            pl.BlockSpec((1, 1), lambda t: (0, 0)),
        ],
        out_specs=pl.BlockSpec((N_MOLS, 1), lambda t: (0, 0)),
        out_shape=jax.ShapeDtypeStruct((N_MOLS, 1), jnp.float32),
        scratch_shapes=[
            pltpu.VMEM((N_MOLS, GRU_H), jnp.float32),
            pltpu.VMEM((N_MOLS, HID), jnp.float32),
        ],
    )(msgT, h0, wihT, whhT, bih, bhh, wo, bo, w1, b1, w2, b2)


def kernel(f_atoms, f_bonds, a2b, b2a, b2revb, W_i_atom, W_i_bond, W_h_bond,
           W_o, b_o, gru_bias, W_ih, W_hh, b_ih, b_hh, W1, b1, W2, b2):
    f32 = jnp.float32
    fa_p = jnp.pad(f_atoms, ((0, NA_PAD - N_ATOMS), (0, 0)))
    fb_p = jnp.pad(f_bonds, ((0, NB_PAD - N_BONDS), (0, 0)))
    a2b_flat = jnp.pad(a2b.astype(jnp.int32),
                       ((0, NA_PAD - N_ATOMS), (0, 0))).reshape(-1)
    b2a_p = jnp.pad(b2a.astype(jnp.int32), (0, NB_PAD - N_BONDS))
    b2revb_p = jnp.pad(b2revb.astype(jnp.int32), (0, NB_PAD - N_BONDS))

    input_atom = _tc_matmul_relu(fa_p, W_i_atom.astype(f32))
    input_bond = _tc_matmul_relu(fb_p, W_i_bond.astype(f32))

    mb = input_bond
    for _ in range(DEPTH - 1):
        aggp = _sc_aggregate(mb, a2b_flat)
        delta = _sc_combine(aggp, mb, b2a_p, b2revb_p)
        mb = _tc_update(delta, input_bond, W_h_bond.astype(f32))
    aggp = _sc_aggregate(mb, a2b_flat)

    agg1 = aggp[1:N_ATOMS].reshape(N_MOLS, MOL_SIZE, HID)
    ia1 = input_atom[1:N_ATOMS].reshape(N_MOLS, MOL_SIZE, HID)
    fa1 = f_atoms[1:N_ATOMS].reshape(N_MOLS, MOL_SIZE, ATOM_DIM)
    gb = gru_bias.astype(f32).reshape(1, GRU_H)

    msg_seq, h0 = _tc_node(agg1, ia1, fa1, gb)
    msgT = jnp.swapaxes(msg_seq, 0, 1)
    h0 = h0.reshape(N_MOLS, GRU_H)

    out = _tc_gru(
        msgT, h0,
        W_ih.astype(f32).T, W_hh.astype(f32).T,
        b_ih.astype(f32).reshape(1, -1), b_hh.astype(f32).reshape(1, -1),
        W_o.astype(f32), b_o.astype(f32).reshape(1, -1),
        W1.astype(f32), b1.astype(f32).reshape(1, -1),
        W2.astype(f32), b2.astype(f32).reshape(1, -1))
    return out


# speedup vs baseline: 1.3117x; 1.0060x over previous
import functools

import jax
import jax.numpy as jnp
from jax import lax
from jax.experimental import pallas as pl
from jax.experimental.pallas import tpu as pltpu
from jax.experimental.pallas import tpu_sc as plsc

N_MOLS = 100
MOL_SIZE = 100
N_ATOMS = N_MOLS * MOL_SIZE + 1
MAX_NB = 16
N_BONDS = 160001
ATOM_DIM = 128
BOND_DIM = 16
HID = 256
DEPTH = 3
GRU_H = ATOM_DIM + HID

NW = 32
NA_PAD = 10240
NB_PAD = 163840
BM = 2048


def _tc_matmul_relu(x, w, out_dtype=jnp.float32):
    m, k = x.shape
    _, n = w.shape

    def body(x_ref, w_ref, o_ref):
        r = jax.nn.relu(
            jnp.dot(x_ref[...], w_ref[...], preferred_element_type=jnp.float32))
        o_ref[...] = r.astype(out_dtype)

    return pl.pallas_call(
        body,
        grid=(m // BM,),
        in_specs=[
            pl.BlockSpec((BM, k), lambda i: (i, 0)),
            pl.BlockSpec((k, n), lambda i: (0, 0)),
        ],
        out_specs=pl.BlockSpec((BM, n), lambda i: (i, 0)),
        out_shape=jax.ShapeDtypeStruct((m, n), out_dtype),
    )(x, w)


def _tc_proj_bond(x, w):
    m, k = x.shape
    _, n = w.shape

    def body(x_ref, w_ref, o32_ref, o16_ref):
        r = jax.nn.relu(
            jnp.dot(x_ref[...], w_ref[...], preferred_element_type=jnp.float32))
        o32_ref[...] = r
        o16_ref[...] = r.astype(jnp.bfloat16)

    return pl.pallas_call(
        body,
        grid=(m // BM,),
        in_specs=[
            pl.BlockSpec((BM, k), lambda i: (i, 0)),
            pl.BlockSpec((k, n), lambda i: (0, 0)),
        ],
        out_specs=[
            pl.BlockSpec((BM, n), lambda i: (i, 0)),
            pl.BlockSpec((BM, n), lambda i: (i, 0)),
        ],
        out_shape=[
            jax.ShapeDtypeStruct((m, n), jnp.float32),
            jax.ShapeDtypeStruct((m, n), jnp.bfloat16),
        ],
    )(x, w)


def _tc_update(delta, ib16, w16):
    m, k = delta.shape
    _, n = w16.shape

    def body(d_ref, ib_ref, w_ref, o_ref):
        o_ref[...] = jax.nn.relu(
            ib_ref[...].astype(jnp.float32)
            + jnp.dot(d_ref[...].astype(jnp.bfloat16), w_ref[...],
                      preferred_element_type=jnp.float32))

    return pl.pallas_call(
        body,
        grid=(m // BM,),
        in_specs=[
            pl.BlockSpec((BM, k), lambda i: (i, 0)),
            pl.BlockSpec((BM, n), lambda i: (i, 0)),
            pl.BlockSpec((k, n), lambda i: (0, 0)),
        ],
        out_specs=pl.BlockSpec((BM, n), lambda i: (i, 0)),
        out_shape=jax.ShapeDtypeStruct((m, n), jnp.float32),
    )(delta, ib16, w16)


_CA = 8
_CB = 64


_FAST_CORE = 0
_AGG_SPLIT = (432, 208)
_CMB_SPLIT = (6912, 3328)


def _sc_aggregate(mb, a2b_flat):
    per_s = NA_PAD // 16
    g_rows = _CA * MAX_NB
    len_f, len_s = _AGG_SPLIT
    mesh = plsc.VectorSubcoreMesh(core_axis_name="c", subcore_axis_name="s")

    @functools.partial(
        pl.kernel, mesh=mesh,
        out_type=jax.ShapeDtypeStruct((NA_PAD, HID), jnp.float32),
        scratch_types=[
            pltpu.VMEM((per_s * MAX_NB,), jnp.int32),
            pltpu.VMEM((2, g_rows, HID), jnp.float32),
            pltpu.VMEM((_CA, HID), jnp.float32),
            pltpu.SemaphoreType.DMA,
            pltpu.SemaphoreType.DMA,
        ],
    )
    def k(mb_hbm, a2b_hbm, agg_hbm, a2b_l, rows_v, outb_v, sem0, sem1):
        s_idx = lax.axis_index("s")
        c_idx = lax.axis_index("c")
        base_s = s_idx * per_s
        sems = (sem0, sem1)
        pltpu.sync_copy(a2b_hbm.at[pl.ds(base_s * MAX_NB, per_s * MAX_NB)],
                        a2b_l)

        def run(off, n_atoms):
            n_chunks = n_atoms // _CA

            def gstart(c, b):
                lo = (off + c * _CA) * MAX_NB
                pltpu.async_copy(mb_hbm.at[a2b_l.at[pl.ds(lo, g_rows)]],
                                 rows_v.at[b], sems[b])

            def gwait(b):
                pltpu.make_async_copy(mb_hbm.at[a2b_l.at[pl.ds(0, g_rows)]],
                                      rows_v.at[b], sems[b]).wait()

            def compute_wb(c, b):
                rb = rows_v.at[b]

                def atom_body(a, acc):
                    r0 = a * MAX_NB
                    for col in range(HID // 16):
                        cs = pl.ds(col * 16, 16)
                        v = rb[r0, cs]
                        s = v
                        mx = v
                        for r in range(1, MAX_NB):
                            v = rb[r0 + r, cs]
                            s = s + v
                            mx = jnp.maximum(mx, v)
                        outb_v[a, cs] = s * mx
                    return acc

                lax.fori_loop(0, _CA, atom_body, 0)
                pltpu.sync_copy(
                    outb_v, agg_hbm.at[pl.ds(base_s + off + c * _CA, _CA)])

            gstart(0, 0)

            def pair_body(p, carry):
                c0 = p * 2
                gstart(c0 + 1, 1)
                gwait(0)
                compute_wb(c0, 0)

                @pl.when(c0 + 2 < n_chunks)
                def _():
                    gstart(c0 + 2, 0)

                gwait(1)
                compute_wb(c0 + 1, 1)
                return carry

            lax.fori_loop(0, n_chunks // 2, pair_body, 0)

        @pl.when(c_idx == _FAST_CORE)
        def _():
            run(0, len_f)

        @pl.when(c_idx != _FAST_CORE)
        def _():
            run(len_f, len_s)

    return k(mb, a2b_flat)


def _sc_combine(agg, mb, b2a, b2revb):
    per_s = NB_PAD // 16
    len_f, len_s = _CMB_SPLIT
    mesh = plsc.VectorSubcoreMesh(core_axis_name="c", subcore_axis_name="s")

    @functools.partial(
        pl.kernel, mesh=mesh,
        out_type=jax.ShapeDtypeStruct((NB_PAD, HID), jnp.float32),
        scratch_types=[
            pltpu.VMEM((per_s,), jnp.int32),
            pltpu.VMEM((per_s,), jnp.int32),
            pltpu.VMEM((2, _CB, HID), jnp.float32),
            pltpu.VMEM((2, _CB, HID), jnp.float32),
            pltpu.SemaphoreType.DMA,
            pltpu.SemaphoreType.DMA,
            pltpu.SemaphoreType.DMA,
            pltpu.SemaphoreType.DMA,
        ],
    )
    def k(agg_hbm, mb_hbm, b2a_hbm, b2revb_hbm, out_hbm,
          b2a_l, b2revb_l, g1_v, g2_v, s1a, s1b, s2a, s2b):
        s_idx = lax.axis_index("s")
        c_idx = lax.axis_index("c")
        base_s = s_idx * per_s
        s1 = (s1a, s1b)
        s2 = (s2a, s2b)
        pltpu.sync_copy(b2a_hbm.at[pl.ds(base_s, per_s)], b2a_l)
        pltpu.sync_copy(b2revb_hbm.at[pl.ds(base_s, per_s)], b2revb_l)

        def run(off, n_bonds):
            n_chunks = n_bonds // _CB

            def gstart(c, b):
                sl = pl.ds(off + c * _CB, _CB)
                pltpu.async_copy(agg_hbm.at[b2a_l.at[sl]], g1_v.at[b], s1[b])
                pltpu.async_copy(mb_hbm.at[b2revb_l.at[sl]], g2_v.at[b], s2[b])

            def gwait(b):
                sl = pl.ds(0, _CB)
                pltpu.make_async_copy(agg_hbm.at[b2a_l.at[sl]], g1_v.at[b],
                                      s1[b]).wait()
                pltpu.make_async_copy(mb_hbm.at[b2revb_l.at[sl]], g2_v.at[b],
                                      s2[b]).wait()

            def compute_wb(c, b):
                gb1 = g1_v.at[b]
                gb2 = g2_v.at[b]

                def row_body(r, acc):
                    for col in range(HID // 16):
                        cs = pl.ds(col * 16, 16)
                        gb1[r, cs] = gb1[r, cs] - gb2[r, cs]
                    return acc

                lax.fori_loop(0, _CB, row_body, 0)
                pltpu.sync_copy(gb1,
                                out_hbm.at[pl.ds(base_s + off + c * _CB, _CB)])

            gstart(0, 0)

            def pair_body(p, carry):
                c0 = p * 2
                gstart(c0 + 1, 1)
                gwait(0)
                compute_wb(c0, 0)

                @pl.when(c0 + 2 < n_chunks)
                def _():
                    gstart(c0 + 2, 0)

                gwait(1)
                compute_wb(c0 + 1, 1)
                return carry

            lax.fori_loop(0, n_chunks // 2, pair_body, 0)

        @pl.when(c_idx == _FAST_CORE)
        def _():
            run(0, len_f)

        @pl.when(c_idx != _FAST_CORE)
        def _():
            run(len_f, len_s)

    return k(agg, mb, b2a, b2revb)


def _tc_node(agg1, ia1, fa1, gb):
    def body(a_ref, i_ref, f_ref, gb_ref, msg_ref, h0_ref):
        a = a_ref[0].astype(jnp.float32)
        node = jnp.concatenate([a * i_ref[0], f_ref[0]], axis=1)
        msg_ref[...] = jax.nn.relu(node + gb_ref[...]).reshape(1, MOL_SIZE, GRU_H)
        h0_ref[...] = jnp.max(node, axis=0).reshape(1, 1, GRU_H)

    return pl.pallas_call(
        body,
        grid=(N_MOLS,),
        in_specs=[
            pl.BlockSpec((1, MOL_SIZE, HID), lambda m: (m, 0, 0)),
            pl.BlockSpec((1, MOL_SIZE, HID), lambda m: (m, 0, 0)),
            pl.BlockSpec((1, MOL_SIZE, ATOM_DIM), lambda m: (m, 0, 0)),
            pl.BlockSpec((1, GRU_H), lambda m: (0, 0)),
        ],
        out_specs=[
            pl.BlockSpec((1, MOL_SIZE, GRU_H), lambda m: (m, 0, 0)),
            pl.BlockSpec((1, 1, GRU_H), lambda m: (m, 0, 0)),
        ],
        out_shape=[
            jax.ShapeDtypeStruct((N_MOLS, MOL_SIZE, GRU_H), jnp.float32),
            jax.ShapeDtypeStruct((N_MOLS, 1, GRU_H), jnp.float32),
        ],
    )(agg1, ia1, fa1, gb)


def _tc_gru(msgT, h0, wihT, whhT, bih, bhh, wo, bo, w1, b1, w2, b2):
    def body(x_ref, h0_ref, wih_ref, whh_ref, bih_ref, bhh_ref, wo_ref, bo_ref,
             w1_ref, b1_ref, w2_ref, b2_ref, out_ref, h_ref, mv_ref):
        t = pl.program_id(0)

        @pl.when(t == 0)
        def _():
            h_ref[...] = h0_ref[...]

        x = x_ref[0]
        h = h_ref[...]
        gx = jnp.dot(x, wih_ref[...], preferred_element_type=jnp.float32) + bih_ref[...]
        gh = jnp.dot(h, whh_ref[...], preferred_element_type=jnp.float32) + bhh_ref[...]
        r = jax.nn.sigmoid(gx[:, :GRU_H] + gh[:, :GRU_H])
        z = jax.nn.sigmoid(gx[:, GRU_H:2 * GRU_H] + gh[:, GRU_H:2 * GRU_H])
        n = jnp.tanh(gx[:, 2 * GRU_H:] + r * gh[:, 2 * GRU_H:])
        h_new = (1.0 - z) * n + z * h
        h_ref[...] = h_new
        a = jax.nn.relu(
            jnp.dot(h_new, wo_ref[...], preferred_element_type=jnp.float32)
            + bo_ref[...])

        @pl.when(t == 0)
        def _():
            mv_ref[...] = a

        @pl.when(t > 0)
        def _():
            mv_ref[...] = jnp.maximum(mv_ref[...], a)

        @pl.when(t == N_MOLS - 1)
        def _():
            hf = jax.nn.relu(
                jnp.dot(mv_ref[...], w1_ref[...], preferred_element_type=jnp.float32)
                + b1_ref[...])
            out_ref[...] = (
                jnp.dot(hf, w2_ref[...], preferred_element_type=jnp.float32)
                + b2_ref[...])

    return pl.pallas_call(
        body,
        grid=(MOL_SIZE,),
        in_specs=[
            pl.BlockSpec((1, N_MOLS, GRU_H), lambda t: (t, 0, 0)),
            pl.BlockSpec((N_MOLS, GRU_H), lambda t: (0, 0)),
            pl.BlockSpec((GRU_H, 3 * GRU_H), lambda t: (0, 0)),
            pl.BlockSpec((GRU_H, 3 * GRU_H), lambda t: (0, 0)),
            pl.BlockSpec((1, 3 * GRU_H), lambda t: (0, 0)),
            pl.BlockSpec((1, 3 * GRU_H), lambda t: (0, 0)),
            pl.BlockSpec((GRU_H, HID), lambda t: (0, 0)),
            pl.BlockSpec((1, HID), lambda t: (0, 0)),
            pl.BlockSpec((HID, HID), lambda t: (0, 0)),
            pl.BlockSpec((1, HID), lambda t: (0, 0)),
            pl.BlockSpec((HID, 1), lambda t: (0, 0)),
            pl.BlockSpec((1, 1), lambda t: (0, 0)),
        ],
        out_specs=pl.BlockSpec((N_MOLS, 1), lambda t: (0, 0)),
        out_shape=jax.ShapeDtypeStruct((N_MOLS, 1), jnp.float32),
        scratch_shapes=[
            pltpu.VMEM((N_MOLS, GRU_H), jnp.float32),
            pltpu.VMEM((N_MOLS, HID), jnp.float32),
        ],
    )(msgT, h0, wihT, whhT, bih, bhh, wo, bo, w1, b1, w2, b2)


def kernel(f_atoms, f_bonds, a2b, b2a, b2revb, W_i_atom, W_i_bond, W_h_bond,
           W_o, b_o, gru_bias, W_ih, W_hh, b_ih, b_hh, W1, b1, W2, b2):
    f32 = jnp.float32
    fa_p = jnp.pad(f_atoms, ((0, NA_PAD - N_ATOMS), (0, 0)))
    fb_p = jnp.pad(f_bonds, ((0, NB_PAD - N_BONDS), (0, 0)))
    a2b_flat = jnp.pad(a2b.astype(jnp.int32),
                       ((0, NA_PAD - N_ATOMS), (0, 0))).reshape(-1)
    b2a_p = jnp.pad(b2a.astype(jnp.int32), (0, NB_PAD - N_BONDS))
    b2revb_p = jnp.pad(b2revb.astype(jnp.int32), (0, NB_PAD - N_BONDS))

    input_atom = _tc_matmul_relu(fa_p, W_i_atom.astype(f32))
    input_bond, input_bond16 = _tc_proj_bond(fb_p, W_i_bond.astype(f32))
    w_h16 = W_h_bond.astype(jnp.bfloat16)

    mb = input_bond
    for _ in range(DEPTH - 1):
        aggp = _sc_aggregate(mb, a2b_flat)
        delta = _sc_combine(aggp, mb, b2a_p, b2revb_p)
        mb = _tc_update(delta, input_bond16, w_h16)
    aggp = _sc_aggregate(mb, a2b_flat)

    agg1 = aggp[1:N_ATOMS].reshape(N_MOLS, MOL_SIZE, HID)
    ia1 = input_atom[1:N_ATOMS].reshape(N_MOLS, MOL_SIZE, HID)
    fa1 = f_atoms[1:N_ATOMS].reshape(N_MOLS, MOL_SIZE, ATOM_DIM)
    gb = gru_bias.astype(f32).reshape(1, GRU_H)

    msg_seq, h0 = _tc_node(agg1, ia1, fa1, gb)
    msgT = jnp.swapaxes(msg_seq, 0, 1)
    h0 = h0.reshape(N_MOLS, GRU_H)

    out = _tc_gru(
        msgT, h0,
        W_ih.astype(f32).T, W_hh.astype(f32).T,
        b_ih.astype(f32).reshape(1, -1), b_hh.astype(f32).reshape(1, -1),
        W_o.astype(f32), b_o.astype(f32).reshape(1, -1),
        W1.astype(f32), b1.astype(f32).reshape(1, -1),
        W2.astype(f32), b2.astype(f32).reshape(1, -1))
    return out
